# baseline (speedup 1.0000x reference)
import functools

import jax
import jax.numpy as jnp
from jax import lax
from jax.experimental import pallas as pl
from jax.experimental.pallas import tpu as pltpu
from jax.experimental.pallas import tpu_sc as plsc

N_CORES = 2
N_SUB = 16
N_WORKERS = N_CORES * N_SUB
LANES = 16
SB = 112


@functools.lru_cache(maxsize=None)
def _build_sc_aggregate(n, d, e_pad):
    per_worker = e_pad // N_WORKERS
    assert per_worker * N_WORKERS == e_pad
    n_sb = per_worker // SB
    assert n_sb * SB == per_worker and n_sb >= 3
    rows_per_tile = n // N_SUB
    zchunk = 80
    n_zc = rows_per_tile // zchunk
    assert n_zc * zchunk == rows_per_tile and zchunk <= SB

    mesh = plsc.VectorSubcoreMesh(core_axis_name="c", subcore_axis_name="s")

    NBUF = 3

    @functools.partial(
        pl.kernel,
        out_type=jax.ShapeDtypeStruct((N_CORES, n, d), jnp.float32),
        mesh=mesh,
        scratch_types=(
            [pltpu.VMEM((SB,), jnp.int32) for _ in range(NBUF)]
            + [pltpu.VMEM((SB,), jnp.float32) for _ in range(NBUF)]
            + [pltpu.VMEM((SB,), jnp.int32) for _ in range(NBUF)]
            + [pltpu.VMEM((SB, d), jnp.float32) for _ in range(NBUF)]
            + [pltpu.VMEM_SHARED((n, d), jnp.float32)]
            + [pltpu.SemaphoreType.DMA for _ in range(4 * NBUF)]
        ),
    )
    def sc_aggr(h_hbm, src_hbm, wgt_hbm, dst_hbm, out_hbm,
                sw0, sw1, sw2, wv0, wv1, wv2, dst0, dst1, dst2,
                rows0, rows1, rows2, acc_sp, *sems):
        zbuf = rows0
        c = lax.axis_index("c")
        s = lax.axis_index("s")
        wid = c * N_SUB + s
        sw = (sw0, sw1, sw2)
        wv = (wv0, wv1, wv2)
        dst = (dst0, dst1, dst2)
        rows = (rows0, rows1, rows2)
        wsem = sems[0:NBUF]
        dsem = sems[NBUF:2 * NBUF]
        gsem = sems[2 * NBUF:3 * NBUF]
        ssem = sems[3 * NBUF:4 * NBUF]
        sb_base = wid * n_sb
        ebase = wid * per_worker

        def z_body(i, _):
            for j in range(d // LANES):
                zbuf[i, pl.ds(j * LANES, LANES)] = jnp.zeros((LANES,), jnp.float32)
            return 0
        lax.fori_loop(0, zchunk, z_body, 0)
        row0 = s * rows_per_tile
        for k in range(n_zc):
            pltpu.sync_copy(zbuf.at[pl.ds(0, zchunk)],
                            acc_sp.at[pl.ds(row0 + k * zchunk, zchunk)])
        plsc.subcore_barrier()

        def start_idx(i, q):
            pltpu.async_copy(src_hbm.at[pl.ds(ebase + i * SB, SB)],
                             sw[q], wsem[q])
            pltpu.async_copy(wgt_hbm.at[pl.ds(ebase + i * SB, SB)],
                             wv[q], wsem[q])
            pltpu.async_copy(dst_hbm.at[pl.ds(ebase + i * SB, SB)],
                             dst[q], dsem[q])

        def wait_idx(q):
            pltpu.make_async_copy(src_hbm.at[pl.ds(0, SB)], sw[q],
                                  wsem[q]).wait()
            pltpu.make_async_copy(wgt_hbm.at[pl.ds(0, SB)], wv[q],
                                  wsem[q]).wait()

        def wait_dst(q):
            pltpu.make_async_copy(dst_hbm.at[pl.ds(0, SB)], dst[q],
                                  dsem[q]).wait()

        def start_gather(q):
            pltpu.async_copy(h_hbm.at[sw[q]], rows[q], gsem[q])

        def drain_gather(q):
            pltpu.make_async_copy(h_hbm.at[sw[q]], rows[q],
                                  gsem[q]).wait()

        def scale(q):
            rows_v = rows[q]

            for g in range(SB // LANES):
                gb = g * LANES
                w16 = wv[q][pl.ds(gb, LANES)]
                for b in range(LANES):
                    wb = lax.gather(
                        w16, jnp.full((LANES, 1), b, jnp.int32),
                        lax.GatherDimensionNumbers(
                            offset_dims=(), collapsed_slice_dims=(0,),
                            start_index_map=(0,)),
                        slice_sizes=(1,),
                        mode=lax.GatherScatterMode.PROMISE_IN_BOUNDS)
                    for j in range(d // LANES):
                        sl = pl.ds(j * LANES, LANES)
                        rows_v[gb + b, sl] = rows_v[gb + b, sl] * wb

        def start_scatter(q):
            pltpu.async_copy(rows[q], acc_sp.at[dst[q]], ssem[q], add=True)

        def drain_scatter(q):
            pltpu.make_async_copy(rows[q], acc_sp.at[dst[q]], ssem[q]).wait()

        start_idx(0, 0)
        start_idx(1, 1)
        wait_idx(0)
        start_gather(0)
        wait_idx(1)
        start_gather(1)

        def sb_body(i, _):
            p = lax.rem(i, NBUF)
            for q in range(NBUF):
                @pl.when(p == q)
                def _():
                    r = (q + 2) % NBUF

                    @pl.when(i + 2 <= n_sb - 1)
                    def _():
                        start_idx(i + 2, r)
                    drain_gather(q)
                    scale(q)
                    wait_dst(q)
                    start_scatter(q)

                    @pl.when(i + 2 <= n_sb - 1)
                    def _():
                        @pl.when(i >= 1)
                        def _():
                            drain_scatter(r)
                        wait_idx(r)
                        start_gather(r)
            return 0
        lax.fori_loop(0, n_sb, sb_body, 0)
        for k in range(3):
            drain_scatter((n_sb - 3 + k) % NBUF)

        plsc.subcore_barrier()
        pltpu.sync_copy(acc_sp.at[pl.ds(row0, rows_per_tile)],
                        out_hbm.at[c, pl.ds(row0, rows_per_tile)])

    return sc_aggr


@functools.lru_cache(maxsize=None)
def _build_tc_update(n, d):
    blk = 1024
    assert n % blk == 0

    def tc_body(a0_ref, a1_ref, h_ref, w1t_ref, w2t_ref, b_ref, out_ref):
        aggr = a0_ref[...] + a1_ref[...]
        z = jnp.dot(aggr, w1t_ref[...], preferred_element_type=jnp.float32)
        z = z + jnp.dot(h_ref[...], w2t_ref[...],
                        preferred_element_type=jnp.float32)
        out_ref[...] = jnp.tanh(z + b_ref[...])

    return pl.pallas_call(
        tc_body,
        grid=(n // blk,),
        in_specs=[
            pl.BlockSpec((blk, d), lambda i: (i, 0)),
            pl.BlockSpec((blk, d), lambda i: (i, 0)),
            pl.BlockSpec((blk, d), lambda i: (i, 0)),
            pl.BlockSpec((d, d), lambda i: (0, 0)),
            pl.BlockSpec((d, d), lambda i: (0, 0)),
            pl.BlockSpec((1, d), lambda i: (0, 0)),
        ],
        out_specs=pl.BlockSpec((blk, d), lambda i: (i, 0)),
        out_shape=jax.ShapeDtypeStruct((n, d), jnp.float32),
    )


def kernel(x, edge_index, edge_weights, Ws, bs):
    n, d = x.shape
    e = edge_weights.shape[0]
    num_layers = Ws.shape[0]

    n_pad = ((n + 2047) // 2048) * 2048
    chunk = N_WORKERS * SB
    e_pad = ((e + chunk - 1) // chunk) * chunk

    fill = (jnp.arange(e_pad - e, dtype=jnp.int32) % n_pad)
    src = jnp.concatenate([edge_index[0], fill])
    dst = jnp.concatenate([edge_index[1], fill])
    ew = jnp.concatenate(
        [edge_weights, jnp.zeros((e_pad - e,), jnp.float32)])


    w1t = jnp.transpose(Ws[:, :, :d], (0, 2, 1))
    w2t = jnp.transpose(Ws[:, :, d:], (0, 2, 1))
    b2 = bs.reshape(num_layers, 1, d)

    sc_aggr = _build_sc_aggregate(n_pad, d, e_pad)
    tc_update = _build_tc_update(n_pad, d)

    h = jnp.pad(x, ((0, n_pad - n), (0, 0)))
    for l in range(num_layers):
        parts = sc_aggr(h, src, ew, dst)
        h = tc_update(parts[0], parts[1], h, w1t[l], w2t[l], b2[l])
    return h[:n]


# speedup vs baseline: 1.3852x; 1.3852x over previous
import functools

import jax
import jax.numpy as jnp
from jax import lax
from jax.experimental import pallas as pl
from jax.experimental.pallas import tpu as pltpu
from jax.experimental.pallas import tpu_sc as plsc

N_CORES = 2
N_SUB = 16
N_WORKERS = N_CORES * N_SUB
LANES = 16
SB = 112


@functools.lru_cache(maxsize=None)
def _build_sc_aggregate(n, d, e_pad):
    per_worker = e_pad // N_WORKERS
    assert per_worker * N_WORKERS == e_pad
    n_sb = per_worker // SB
    assert n_sb * SB == per_worker and n_sb >= 3
    rows_per_tile = n // N_SUB
    zchunk = 80
    n_zc = rows_per_tile // zchunk
    assert n_zc * zchunk == rows_per_tile and zchunk <= SB

    mesh = plsc.VectorSubcoreMesh(core_axis_name="c", subcore_axis_name="s")

    NBUF = 3

    @functools.partial(
        pl.kernel,
        out_type=jax.ShapeDtypeStruct((N_CORES, n, d), jnp.float32),
        mesh=mesh,
        scratch_types=(
            [pltpu.VMEM((SB,), jnp.int32) for _ in range(NBUF)]
            + [pltpu.VMEM((SB,), jnp.float32) for _ in range(NBUF)]
            + [pltpu.VMEM((SB,), jnp.int32) for _ in range(NBUF)]
            + [pltpu.VMEM((SB, d), jnp.float32) for _ in range(NBUF)]
            + [pltpu.VMEM_SHARED((n, d), jnp.float32)]
            + [pltpu.SemaphoreType.DMA for _ in range(4 * NBUF)]
        ),
    )
    def sc_aggr(h_hbm, src_hbm, wgt_hbm, dst_hbm, out_hbm,
                sw0, sw1, sw2, wv0, wv1, wv2, dst0, dst1, dst2,
                rows0, rows1, rows2, acc_sp, *sems):
        zbuf = rows0
        c = lax.axis_index("c")
        s = lax.axis_index("s")
        wid = c * N_SUB + s
        sw = (sw0, sw1, sw2)
        wv = (wv0, wv1, wv2)
        dst = (dst0, dst1, dst2)
        rows = (rows0, rows1, rows2)
        wsem = sems[0:NBUF]
        dsem = sems[NBUF:2 * NBUF]
        gsem = sems[2 * NBUF:3 * NBUF]
        ssem = sems[3 * NBUF:4 * NBUF]
        sb_base = wid * n_sb
        ebase = wid * per_worker

        def z_body(i, _):
            for j in range(d // LANES):
                zbuf[i, pl.ds(j * LANES, LANES)] = jnp.zeros((LANES,), jnp.float32)
            return 0
        lax.fori_loop(0, zchunk, z_body, 0)
        row0 = s * rows_per_tile
        for k in range(n_zc):
            pltpu.sync_copy(zbuf.at[pl.ds(0, zchunk)],
                            acc_sp.at[pl.ds(row0 + k * zchunk, zchunk)])
        plsc.subcore_barrier()

        def start_idx(i, q):
            pltpu.async_copy(src_hbm.at[pl.ds(ebase + i * SB, SB)],
                             sw[q], wsem[q])
            pltpu.async_copy(wgt_hbm.at[pl.ds(ebase + i * SB, SB)],
                             wv[q], wsem[q])
            pltpu.async_copy(dst_hbm.at[pl.ds(ebase + i * SB, SB)],
                             dst[q], dsem[q])

        def wait_idx(q):
            pltpu.make_async_copy(src_hbm.at[pl.ds(0, SB)], sw[q],
                                  wsem[q]).wait()
            pltpu.make_async_copy(wgt_hbm.at[pl.ds(0, SB)], wv[q],
                                  wsem[q]).wait()

        def wait_dst(q):
            pltpu.make_async_copy(dst_hbm.at[pl.ds(0, SB)], dst[q],
                                  dsem[q]).wait()

        def start_gather(q):
            pltpu.async_copy(h_hbm.at[sw[q]], rows[q], gsem[q])

        def drain_gather(q):
            pltpu.make_async_copy(h_hbm.at[sw[q]], rows[q],
                                  gsem[q]).wait()

        def scale(q):
            rows_v = rows[q]

            def g_body(g, _):
                gb = g * LANES
                w16 = wv[q][pl.ds(gb, LANES)]
                for b in range(LANES):
                    wb = lax.gather(
                        w16, jnp.full((LANES, 1), b, jnp.int32),
                        lax.GatherDimensionNumbers(
                            offset_dims=(), collapsed_slice_dims=(0,),
                            start_index_map=(0,)),
                        slice_sizes=(1,),
                        mode=lax.GatherScatterMode.PROMISE_IN_BOUNDS)
                    for j in range(d // LANES):
                        sl = pl.ds(j * LANES, LANES)
                        rows_v[gb + b, sl] = rows_v[gb + b, sl] * wb
                return 0
            lax.fori_loop(0, SB // LANES, g_body, 0)

        def start_scatter(q):
            pltpu.async_copy(rows[q], acc_sp.at[dst[q]], ssem[q], add=True)

        def drain_scatter(q):
            pltpu.make_async_copy(rows[q], acc_sp.at[dst[q]], ssem[q]).wait()

        start_idx(0, 0)
        start_idx(1, 1)
        wait_idx(0)
        start_gather(0)
        wait_idx(1)
        start_gather(1)

        def sb_body(i, _):
            p = lax.rem(i, NBUF)
            for q in range(NBUF):
                @pl.when(p == q)
                def _():
                    r = (q + 2) % NBUF

                    @pl.when(i + 2 <= n_sb - 1)
                    def _():
                        start_idx(i + 2, r)
                    drain_gather(q)
                    scale(q)
                    wait_dst(q)
                    start_scatter(q)

                    @pl.when(i + 2 <= n_sb - 1)
                    def _():
                        @pl.when(i >= 1)
                        def _():
                            drain_scatter(r)
                        wait_idx(r)
                        start_gather(r)
            return 0
        lax.fori_loop(0, n_sb, sb_body, 0)
        for k in range(3):
            drain_scatter((n_sb - 3 + k) % NBUF)

        plsc.subcore_barrier()
        pltpu.sync_copy(acc_sp.at[pl.ds(row0, rows_per_tile)],
                        out_hbm.at[c, pl.ds(row0, rows_per_tile)])

    return sc_aggr


@functools.lru_cache(maxsize=None)
def _build_tc_update(n, d):
    blk = 1024
    assert n % blk == 0

    def tc_body(a0_ref, a1_ref, h_ref, w1t_ref, w2t_ref, b_ref, out_ref):
        aggr = a0_ref[...] + a1_ref[...]
        z = jnp.dot(aggr, w1t_ref[...], preferred_element_type=jnp.float32)
        z = z + jnp.dot(h_ref[...], w2t_ref[...],
                        preferred_element_type=jnp.float32)
        out_ref[...] = jnp.tanh(z + b_ref[...])

    return pl.pallas_call(
        tc_body,
        grid=(n // blk,),
        in_specs=[
            pl.BlockSpec((blk, d), lambda i: (i, 0)),
            pl.BlockSpec((blk, d), lambda i: (i, 0)),
            pl.BlockSpec((blk, d), lambda i: (i, 0)),
            pl.BlockSpec((d, d), lambda i: (0, 0)),
            pl.BlockSpec((d, d), lambda i: (0, 0)),
            pl.BlockSpec((1, d), lambda i: (0, 0)),
        ],
        out_specs=pl.BlockSpec((blk, d), lambda i: (i, 0)),
        out_shape=jax.ShapeDtypeStruct((n, d), jnp.float32),
    )


def kernel(x, edge_index, edge_weights, Ws, bs):
    n, d = x.shape
    e = edge_weights.shape[0]
    num_layers = Ws.shape[0]

    n_pad = ((n + 2047) // 2048) * 2048
    chunk = N_WORKERS * SB
    e_pad = ((e + chunk - 1) // chunk) * chunk

    fill = (jnp.arange(e_pad - e, dtype=jnp.int32) % n_pad)
    src = jnp.concatenate([edge_index[0], fill])
    dst = jnp.concatenate([edge_index[1], fill])
    ew = jnp.concatenate(
        [edge_weights, jnp.zeros((e_pad - e,), jnp.float32)])


    w1t = jnp.transpose(Ws[:, :, :d], (0, 2, 1))
    w2t = jnp.transpose(Ws[:, :, d:], (0, 2, 1))
    b2 = bs.reshape(num_layers, 1, d)

    sc_aggr = _build_sc_aggregate(n_pad, d, e_pad)
    tc_update = _build_tc_update(n_pad, d)

    h = jnp.pad(x, ((0, n_pad - n), (0, 0)))
    for l in range(num_layers):
        parts = sc_aggr(h, src, ew, dst)
        h = tc_update(parts[0], parts[1], h, w1t[l], w2t[l], b2[l])
    return h[:n]
